# speedup vs baseline: 17.5663x; 17.5663x over previous
import math

import jax
import jax.numpy as jnp
import numpy as np
from jax import lax
from jax.experimental import pallas as pl
from jax.experimental.pallas import tpu as pltpu

_IMAGE_SIZE = 300
_STEPS = [16, 32, 64, 100, 150, 300]
_MIN_SIZES = [60, 105, 150, 195, 240, 285]
_MAX_SIZES = [105, 150, 195, 240, 285, 330]
_ASPECT_RATIOS = [[2, 3]] * 6
_VAR_CENTER = 0.1
_VAR_SIZE = 0.2
_SCORE_THRESH = 0.05
_NMS_THRESH = 0.45
_TOP_N = 100
_NEG = -1e10

_NPRI = 3000
_NPAD = 3072
_CROWS = 24


def _gen_priors_padded():
    priors = []
    for k, step in enumerate(_STEPS):
        f = int(math.ceil(_IMAGE_SIZE / step))
        s_k = _MIN_SIZES[k] / _IMAGE_SIZE
        s_k_prime = math.sqrt(_MIN_SIZES[k] * _MAX_SIZES[k]) / _IMAGE_SIZE
        for i in range(f):
            for j in range(f):
                cx = (j + 0.5) * step / _IMAGE_SIZE
                cy = (i + 0.5) * step / _IMAGE_SIZE
                priors.append([cx, cy, s_k, s_k])
                priors.append([cx, cy, s_k_prime, s_k_prime])
                for ar in _ASPECT_RATIOS[k]:
                    r = math.sqrt(ar)
                    priors.append([cx, cy, s_k * r, s_k / r])
                    priors.append([cx, cy, s_k / r, s_k * r])
    p = np.asarray(priors, dtype=np.float32)
    assert p.shape == (_NPRI, 4)
    pt = np.zeros((8, _NPAD), np.float32)
    pt[0:4, :_NPRI] = p.T
    pt[2, _NPRI:] = 1.0
    pt[3, _NPRI:] = 1.0
    return pt


_PRIORS_T = _gen_priors_padded()


def _nms_kernel(deltas_ref, logits_ref, priors_ref,
                boxes_out, scores_out, labels_out,
                s_ref, box_ref, rmax_ref, kept_ref):
    f32 = jnp.float32
    col1 = lax.broadcasted_iota(jnp.int32, (1, _NPAD), 1)
    valid_col1 = col1 < _NPRI

    d = deltas_ref[0]
    pr = priors_ref[...]
    cx = pr[0:1] + d[0:1] * _VAR_CENTER * pr[2:3]
    cy = pr[1:2] + d[1:2] * _VAR_CENTER * pr[3:4]
    w = pr[2:3] * jnp.exp(d[2:3] * _VAR_SIZE)
    h = pr[3:4] * jnp.exp(d[3:4] * _VAR_SIZE)
    x1 = cx - w / 2.0
    y1 = cy - h / 2.0
    x2 = cx + w / 2.0
    y2 = cy + h / 2.0
    box_ref[0:1, :] = x1
    box_ref[1:2, :] = y1
    box_ref[2:3, :] = x2
    box_ref[3:4, :] = y2
    mx = jnp.maximum(jnp.maximum(x1, y1), jnp.maximum(x2, y2))
    big_m = jnp.max(jnp.where(valid_col1, mx, -jnp.inf)) + 1.0

    lg = logits_ref[0]
    cmax = jnp.max(lg, axis=0, keepdims=True)
    e = jnp.exp(lg - cmax)
    probs = e / jnp.sum(e, axis=0, keepdims=True)
    row24 = lax.broadcasted_iota(jnp.int32, (_CROWS, _NPAD), 0)
    col24 = lax.broadcasted_iota(jnp.int32, (_CROWS, _NPAD), 1)
    valid = (row24 >= 1) & (row24 <= 20) & (col24 < _NPRI)
    s = jnp.where(valid & (probs > _SCORE_THRESH), probs, _NEG)
    s_ref[...] = s
    rmax_ref[...] = jnp.broadcast_to(
        jnp.max(s, axis=1, keepdims=True), (_CROWS, 128))
    kept_ref[...] = jnp.zeros((8, 128), f32)
    boxes_out[0] = jnp.zeros((8, 128), f32)
    scores_out[0] = jnp.zeros((8, 128), f32)
    labels_out[0] = jnp.zeros((8, 128), jnp.int32)

    row24s = lax.broadcasted_iota(jnp.int32, (_CROWS, 128), 0)
    lane8 = lax.broadcasted_iota(jnp.int32, (8, 128), 1)
    row8 = lax.broadcasted_iota(jnp.int32, (8, 128), 0)
    lane1 = lax.broadcasted_iota(jnp.int32, (1, 128), 1)

    def cond_fn(carry):
        return jnp.logical_not(carry[2])

    def body_fn(carry):
        k, it, done = carry
        rm = rmax_ref[...]
        m = jnp.max(rm)
        found = m > _SCORE_THRESH
        r = jnp.min(jnp.where(rm == m, row24s, 10 ** 6))
        srow = s_ref[pl.ds(r, 1), :]
        p = jnp.min(jnp.where(srow == m, col1, 10 ** 6))

        cmask = col1 == p
        bx1 = jnp.max(jnp.where(cmask, box_ref[0:1, :], -jnp.inf))
        by1 = jnp.max(jnp.where(cmask, box_ref[1:2, :], -jnp.inf))
        bx2 = jnp.max(jnp.where(cmask, box_ref[2:3, :], -jnp.inf))
        by2 = jnp.max(jnp.where(cmask, box_ref[3:4, :], -jnp.inf))
        off = r.astype(f32) * big_m
        x1o = bx1 + off
        y1o = by1 + off
        x2o = bx2 + off
        y2o = by2 + off
        area_b = (x2o - x1o) * (y2o - y1o)

        iw = jnp.maximum(
            jnp.minimum(kept_ref[2:3, :], x2o) - jnp.maximum(kept_ref[0:1, :], x1o), 0.0)
        ih = jnp.maximum(
            jnp.minimum(kept_ref[3:4, :], y2o) - jnp.maximum(kept_ref[1:2, :], y1o), 0.0)
        inter = iw * ih
        iou = inter / (kept_ref[4:5, :] + area_b - inter + 1e-9)
        suppressed = jnp.max(jnp.where(lane1 < k, iou, 0.0)) > _NMS_THRESH
        keep = jnp.logical_and(found, jnp.logical_not(suppressed))

        @pl.when(found)
        def _():
            new_row = jnp.where(cmask, _NEG, srow)
            s_ref[pl.ds(r, 1), :] = new_row
            nm = jnp.max(new_row)
            rmax_ref[pl.ds(r, 1), :] = jnp.broadcast_to(
                jnp.reshape(nm, (1, 1)), (1, 128))

        @pl.when(keep)
        def _():
            kvals = jnp.where(row8 == 0, x1o,
                    jnp.where(row8 == 1, y1o,
                    jnp.where(row8 == 2, x2o,
                    jnp.where(row8 == 3, y2o, area_b))))
            kept_ref[...] = jnp.where(lane8 == k, kvals, kept_ref[...])
            bvals = jnp.where(row8 == 0, jnp.clip(bx1, 0.0, 1.0),
                    jnp.where(row8 == 1, jnp.clip(by1, 0.0, 1.0),
                    jnp.where(row8 == 2, jnp.clip(bx2, 0.0, 1.0),
                              jnp.clip(by2, 0.0, 1.0))))
            boxes_out[0] = jnp.where(lane8 == k, bvals, boxes_out[0])
            scores_out[0] = jnp.where((row8 == 0) & (lane8 == k), m,
                                      scores_out[0])
            labels_out[0] = jnp.where((row8 == 0) & (lane8 == k), r,
                                      labels_out[0])

        k2 = k + keep.astype(jnp.int32)
        it2 = it + 1
        done2 = jnp.logical_or(
            jnp.logical_not(found),
            jnp.logical_or(k2 >= _TOP_N, it2 >= 61000))
        return (k2, it2, done2)

    lax.while_loop(cond_fn, body_fn,
                   (jnp.int32(0), jnp.int32(0), jnp.bool_(False)))


def _run_pallas(deltas_t, logits_t, priors):
    b = deltas_t.shape[0]
    return pl.pallas_call(
        _nms_kernel,
        grid=(b,),
        in_specs=[
            pl.BlockSpec((1, 8, _NPAD), lambda i: (i, 0, 0)),
            pl.BlockSpec((1, _CROWS, _NPAD), lambda i: (i, 0, 0)),
            pl.BlockSpec((8, _NPAD), lambda i: (0, 0)),
        ],
        out_specs=[
            pl.BlockSpec((1, 8, 128), lambda i: (i, 0, 0)),
            pl.BlockSpec((1, 8, 128), lambda i: (i, 0, 0)),
            pl.BlockSpec((1, 8, 128), lambda i: (i, 0, 0)),
        ],
        out_shape=[
            jax.ShapeDtypeStruct((b, 8, 128), jnp.float32),
            jax.ShapeDtypeStruct((b, 8, 128), jnp.float32),
            jax.ShapeDtypeStruct((b, 8, 128), jnp.int32),
        ],
        scratch_shapes=[
            pltpu.VMEM((_CROWS, _NPAD), jnp.float32),
            pltpu.VMEM((8, _NPAD), jnp.float32),
            pltpu.VMEM((_CROWS, 128), jnp.float32),
            pltpu.VMEM((8, 128), jnp.float32),
        ],
    )(deltas_t, logits_t, priors)


def kernel(pred_bbox_deltas, objectness, features):
    del features
    b = pred_bbox_deltas.shape[0]
    deltas = pred_bbox_deltas.reshape(b, -1, 4)
    logits = objectness.reshape(b, deltas.shape[1], -1)
    deltas_t = jnp.transpose(deltas, (0, 2, 1))
    deltas_t = jnp.pad(deltas_t, ((0, 0), (0, 4), (0, _NPAD - _NPRI)))
    logits_t = jnp.transpose(logits, (0, 2, 1))
    logits_t = jnp.pad(logits_t, ((0, 0), (0, _CROWS - 21), (0, _NPAD - _NPRI)),
                       constant_values=-1e30)
    priors = jnp.asarray(_PRIORS_T)
    boxes_p, scores_p, labels_p = _run_pallas(deltas_t, logits_t, priors)
    boxes = jnp.transpose(boxes_p[:, 0:4, 0:_TOP_N], (0, 2, 1))
    scores = scores_p[:, 0, 0:_TOP_N]
    labels = labels_p[:, 0, 0:_TOP_N]
    return boxes, scores, labels
